# baseline (speedup 1.0000x reference)
import functools

import jax
import jax.numpy as jnp
from jax import lax
from jax.experimental import pallas as pl
from jax.experimental.pallas import tpu as pltpu
from jax.experimental.pallas import tpu_sc as plsc

N = 10000
F = 128
C = 16
E = 320000
NC, NS = 2, 16
NW = NC * NS

CHUNK = 128
NCHUNK = 80
EPW = NCHUNK * CHUNK
EPAD = NW * EPW

NPAD = 10240
STRIPE = NPAD // NS
ZROWS = 128

_vector_mesh = plsc.VectorSubcoreMesh(core_axis_name="c", subcore_axis_name="s")


@functools.partial(
    pl.kernel,
    mesh=_vector_mesh,
    out_type=jax.ShapeDtypeStruct((NC, NPAD, F), jnp.float32),
    scratch_types=[
        pltpu.VMEM((CHUNK,), jnp.int32),
        pltpu.VMEM((CHUNK,), jnp.int32),
        pltpu.VMEM((CHUNK, F), jnp.float32),
        pltpu.VMEM_SHARED((NPAD, F), jnp.float32),
        pltpu.SemaphoreType.DMA,
        pltpu.SemaphoreType.DMA,
    ],
)
def _sc_aggregate(g_hbm, src_hbm, dst_hbm, zeros_hbm, out_hbm,
                  src_v, dst_v, rows_v, acc_sh, semG, semD):
    cid = lax.axis_index("c")
    sid = lax.axis_index("s")
    wid = cid * NS + sid

    base = sid * STRIPE
    pltpu.sync_copy(zeros_hbm, acc_sh.at[pl.ds(base, STRIPE)])
    plsc.subcore_barrier()

    @pl.loop(0, NCHUNK)
    def _(j):
        pltpu.sync_copy(src_hbm.at[wid, j], src_v)
        pltpu.async_copy(dst_hbm.at[wid, j], dst_v, semD)
        pltpu.async_copy(g_hbm.at[src_v], rows_v, semG).wait()
        pltpu.make_async_copy(dst_hbm.at[wid, j], dst_v, semD).wait()
        pltpu.sync_copy(rows_v, acc_sh.at[dst_v], add=True)

    plsc.subcore_barrier()
    pltpu.sync_copy(acc_sh.at[pl.ds(base, STRIPE)],
                    out_hbm.at[cid, pl.ds(base, STRIPE)])


@functools.partial(
    pl.kernel,
    mesh=_vector_mesh,
    out_type=jax.ShapeDtypeStruct((NC, NPAD, F), jnp.float32),
    scratch_types=[
        pltpu.VMEM((CHUNK,), jnp.int32),
        pltpu.VMEM((CHUNK, F), jnp.float32),
        pltpu.VMEM_SHARED((NPAD, F), jnp.float32),
    ],
)
def _sc_degree(dst_hbm, ones_hbm, zeros_hbm, out_hbm,
               dst_v, ones_v, acc_sh):
    cid = lax.axis_index("c")
    sid = lax.axis_index("s")
    wid = cid * NS + sid
    base = sid * STRIPE
    pltpu.sync_copy(zeros_hbm, acc_sh.at[pl.ds(base, STRIPE)])
    pltpu.sync_copy(ones_hbm, ones_v)
    plsc.subcore_barrier()

    @pl.loop(0, NCHUNK)
    def _(j):
        pltpu.sync_copy(dst_hbm.at[wid, j], dst_v)
        pltpu.sync_copy(ones_v, acc_sh.at[dst_v], add=True)

    plsc.subcore_barrier()
    pltpu.sync_copy(acc_sh.at[pl.ds(base, STRIPE)],
                    out_hbm.at[cid, pl.ds(base, STRIPE)])


BLK = 512


def _tc_first_body(deg_ref, x_ref, w_ref, g_ref, dinv_ref):
    deg = deg_ref[0] + deg_ref[1] + 1.0
    dinv = lax.rsqrt(deg)
    h = jnp.dot(x_ref[...], w_ref[...], preferred_element_type=jnp.float32)
    g_ref[...] = h * dinv[:, :1]
    dinv_ref[...] = dinv[:, :C]


def _tc_mid_body(parts_ref, g_ref, dinv_ref, b_ref, w_ref, gout_ref):
    dinv = dinv_ref[:, :1]
    s = parts_ref[0] + parts_ref[1] + g_ref[...]
    xl = jnp.maximum(s * dinv + b_ref[...], 0.0)
    gout_ref[...] = jnp.dot(xl, w_ref[...],
                            preferred_element_type=jnp.float32) * dinv


def _tc_out_body(parts_ref, g_ref, dinv_ref, b_ref, w_ref, bout_ref, y_ref):
    dinv = dinv_ref[:, :1]
    s = parts_ref[0] + parts_ref[1] + g_ref[...]
    xl = jnp.maximum(s * dinv + b_ref[...], 0.0)
    logits = jnp.dot(xl, w_ref[...],
                     preferred_element_type=jnp.float32) + bout_ref[...]
    m = jnp.max(logits, axis=1, keepdims=True)
    e = jnp.exp(logits - m)
    y_ref[...] = e / jnp.sum(e, axis=1, keepdims=True)


_GRID = (NPAD // BLK,)
_spec_rowsF = pl.BlockSpec((BLK, F), lambda i: (i, 0))
_spec_rowsC = pl.BlockSpec((BLK, C), lambda i: (i, 0))
_spec_partsF = pl.BlockSpec((NC, BLK, F), lambda i: (0, i, 0))
_spec_partsC = pl.BlockSpec((NC, BLK, C), lambda i: (0, i, 0))
_spec_wFF = pl.BlockSpec((F, F), lambda i: (0, 0))
_spec_wFC = pl.BlockSpec((F, C), lambda i: (0, 0))
_spec_bF = pl.BlockSpec((1, F), lambda i: (0, 0))
_spec_bC = pl.BlockSpec((1, C), lambda i: (0, 0))

_tc_first = pl.pallas_call(
    _tc_first_body,
    grid=_GRID,
    in_specs=[_spec_partsF, _spec_rowsF, _spec_wFF],
    out_specs=[_spec_rowsF, _spec_rowsC],
    out_shape=[jax.ShapeDtypeStruct((NPAD, F), jnp.float32),
               jax.ShapeDtypeStruct((NPAD, C), jnp.float32)],
)

_tc_mid = pl.pallas_call(
    _tc_mid_body,
    grid=_GRID,
    in_specs=[_spec_partsF, _spec_rowsF, _spec_rowsC, _spec_bF, _spec_wFF],
    out_specs=_spec_rowsF,
    out_shape=jax.ShapeDtypeStruct((NPAD, F), jnp.float32),
)

_tc_out = pl.pallas_call(
    _tc_out_body,
    grid=_GRID,
    in_specs=[_spec_partsF, _spec_rowsF, _spec_rowsC, _spec_bF, _spec_wFC,
              _spec_bC],
    out_specs=_spec_rowsC,
    out_shape=jax.ShapeDtypeStruct((NPAD, C), jnp.float32),
)


def kernel(x, edge_index, W1, b1, W2, b2, W3, b3, W4, b4, Wout, bout):
    ei = edge_index.astype(jnp.int32)
    pad = EPAD - E
    src = jnp.concatenate([ei[0], jnp.full((pad,), N, jnp.int32)])
    dst = jnp.concatenate([ei[1], jnp.full((pad,), N, jnp.int32)])
    src3 = src.reshape(NW, NCHUNK, CHUNK)
    dst3 = dst.reshape(NW, NCHUNK, CHUNK)

    x_pad = jnp.pad(x, ((0, NPAD - N), (0, 0)))
    zerosF = jnp.zeros((STRIPE, F), jnp.float32)
    onesF = jnp.ones((CHUNK, F), jnp.float32)

    deg_parts = _sc_degree(dst3, onesF, zerosF)
    g, dinv = _tc_first(deg_parts, x_pad, W1)

    for b_prev, W_next in ((b1, W2), (b2, W3), (b3, W4)):
        parts = _sc_aggregate(g, src3, dst3, zerosF)
        g = _tc_mid(parts, g, dinv, b_prev.reshape(1, F), W_next)

    parts = _sc_aggregate(g, src3, dst3, zerosF)
    y = _tc_out(parts, g, dinv, b4.reshape(1, F), Wout, bout.reshape(1, C))
    return y[:N]


# speedup vs baseline: 2.3466x; 2.3466x over previous
import functools

import jax
import jax.numpy as jnp
from jax import lax
from jax.experimental import pallas as pl
from jax.experimental.pallas import tpu as pltpu
from jax.experimental.pallas import tpu_sc as plsc

N = 10000
F = 128
C = 16
E = 320000
NC, NS = 2, 16
NW = NC * NS

CHUNK = 128
NCHUNK = 80
EPW = NCHUNK * CHUNK
EPAD = NW * EPW

NPAD = 10240
STRIPE = NPAD // NS
ZROWS = 128

_vector_mesh = plsc.VectorSubcoreMesh(core_axis_name="c", subcore_axis_name="s")


@functools.partial(
    pl.kernel,
    mesh=_vector_mesh,
    out_type=jax.ShapeDtypeStruct((NC, NPAD, F), jnp.float32),
    scratch_types=[
        pltpu.VMEM((CHUNK,), jnp.int32),
        pltpu.VMEM((CHUNK,), jnp.int32),
        pltpu.VMEM((CHUNK, F), jnp.float32),
        pltpu.VMEM_SHARED((NPAD, F), jnp.float32),
        pltpu.SemaphoreType.DMA,
        pltpu.SemaphoreType.DMA,
    ],
)
def _sc_aggregate(g_hbm, src_hbm, dst_hbm, zeros_hbm, out_hbm,
                  src_v, dst_v, rows_v, acc_sh, semG, semD):
    cid = lax.axis_index("c")
    sid = lax.axis_index("s")
    wid = cid * NS + sid

    base = sid * STRIPE
    pltpu.sync_copy(zeros_hbm, acc_sh.at[pl.ds(base, STRIPE)])
    plsc.subcore_barrier()

    @pl.loop(0, NCHUNK)
    def _(j):
        pltpu.sync_copy(src_hbm.at[wid, j], src_v)
        pltpu.async_copy(dst_hbm.at[wid, j], dst_v, semD)
        pltpu.async_copy(g_hbm.at[src_v], rows_v, semG).wait()
        pltpu.make_async_copy(dst_hbm.at[wid, j], dst_v, semD).wait()
        pltpu.sync_copy(rows_v, acc_sh.at[dst_v], add=True)

    plsc.subcore_barrier()
    pltpu.sync_copy(acc_sh.at[pl.ds(base, STRIPE)],
                    out_hbm.at[cid, pl.ds(base, STRIPE)])


@functools.partial(
    pl.kernel,
    mesh=_vector_mesh,
    out_type=jax.ShapeDtypeStruct((NC, NPAD, F), jnp.float32),
    scratch_types=[
        pltpu.VMEM((CHUNK,), jnp.int32),
        pltpu.VMEM((CHUNK, F), jnp.float32),
        pltpu.VMEM_SHARED((NPAD, F), jnp.float32),
    ],
)
def _sc_degree(dst_hbm, ones_hbm, zeros_hbm, out_hbm,
               dst_v, ones_v, acc_sh):
    cid = lax.axis_index("c")
    sid = lax.axis_index("s")
    wid = cid * NS + sid
    base = sid * STRIPE
    pltpu.sync_copy(zeros_hbm, acc_sh.at[pl.ds(base, STRIPE)])
    pltpu.sync_copy(ones_hbm, ones_v)
    plsc.subcore_barrier()

    @pl.loop(0, NCHUNK)
    def _(j):
        pltpu.sync_copy(dst_hbm.at[wid, j], dst_v)
        pltpu.sync_copy(ones_v, acc_sh.at[dst_v], add=True)

    plsc.subcore_barrier()
    pltpu.sync_copy(acc_sh.at[pl.ds(base, STRIPE)],
                    out_hbm.at[cid, pl.ds(base, STRIPE)])


BLK = 512


def _tc_first_body(deg_ref, x_ref, w_ref, g_ref, dinv_ref):
    deg = deg_ref[0] + deg_ref[1] + 1.0
    dinv = lax.rsqrt(deg)
    h = jnp.dot(x_ref[...], w_ref[...], preferred_element_type=jnp.float32)
    g_ref[...] = h * dinv[:, :1]
    dinv_ref[...] = dinv[:, :C]


def _tc_mid_body(parts_ref, g_ref, dinv_ref, b_ref, w_ref, gout_ref):
    dinv = dinv_ref[:, :1]
    s = parts_ref[0] + parts_ref[1] + g_ref[...]
    xl = jnp.maximum(s * dinv + b_ref[...], 0.0)
    gout_ref[...] = jnp.dot(xl, w_ref[...],
                            preferred_element_type=jnp.float32) * dinv


def _tc_out_body(parts_ref, g_ref, dinv_ref, b_ref, w_ref, bout_ref, y_ref):
    dinv = dinv_ref[:, :1]
    s = parts_ref[0] + parts_ref[1] + g_ref[...]
    xl = jnp.maximum(s * dinv + b_ref[...], 0.0)
    logits = jnp.dot(xl, w_ref[...],
                     preferred_element_type=jnp.float32) + bout_ref[...]
    m = jnp.max(logits, axis=1, keepdims=True)
    e = jnp.exp(logits - m)
    y_ref[...] = e / jnp.sum(e, axis=1, keepdims=True)


_GRID = (NPAD // BLK,)
_spec_rowsF = pl.BlockSpec((BLK, F), lambda i: (i, 0))
_spec_rowsC = pl.BlockSpec((BLK, C), lambda i: (i, 0))
_spec_partsF = pl.BlockSpec((NC, BLK, F), lambda i: (0, i, 0))
_spec_partsC = pl.BlockSpec((NC, BLK, C), lambda i: (0, i, 0))
_spec_wFF = pl.BlockSpec((F, F), lambda i: (0, 0))
_spec_wFC = pl.BlockSpec((F, C), lambda i: (0, 0))
_spec_bF = pl.BlockSpec((1, F), lambda i: (0, 0))
_spec_bC = pl.BlockSpec((1, C), lambda i: (0, 0))

_tc_first = pl.pallas_call(
    _tc_first_body,
    grid=_GRID,
    in_specs=[_spec_partsF, _spec_rowsF, _spec_wFF],
    out_specs=[_spec_rowsF, _spec_rowsC],
    out_shape=[jax.ShapeDtypeStruct((NPAD, F), jnp.float32),
               jax.ShapeDtypeStruct((NPAD, C), jnp.float32)],
)

_tc_mid = pl.pallas_call(
    _tc_mid_body,
    grid=_GRID,
    in_specs=[_spec_partsF, _spec_rowsF, _spec_rowsC, _spec_bF, _spec_wFF],
    out_specs=_spec_rowsF,
    out_shape=jax.ShapeDtypeStruct((NPAD, F), jnp.float32),
)

_tc_out = pl.pallas_call(
    _tc_out_body,
    grid=_GRID,
    in_specs=[_spec_partsF, _spec_rowsF, _spec_rowsC, _spec_bF, _spec_wFC,
              _spec_bC],
    out_specs=_spec_rowsC,
    out_shape=jax.ShapeDtypeStruct((NPAD, C), jnp.float32),
)


def kernel(x, edge_index, W1, b1, W2, b2, W3, b3, W4, b4, Wout, bout):
    ei = edge_index.astype(jnp.int32)
    pad = EPAD - E
    spare = N + jnp.arange(pad, dtype=jnp.int32) % (NPAD - N)
    src = jnp.concatenate([ei[0], spare])
    dst = jnp.concatenate([ei[1], spare])
    src3 = src.reshape(NW, NCHUNK, CHUNK)
    dst3 = dst.reshape(NW, NCHUNK, CHUNK)

    x_pad = jnp.pad(x, ((0, NPAD - N), (0, 0)))
    zerosF = jnp.zeros((STRIPE, F), jnp.float32)
    onesF = jnp.ones((CHUNK, F), jnp.float32)

    deg_parts = _sc_degree(dst3, onesF, zerosF)
    g, dinv = _tc_first(deg_parts, x_pad, W1)

    for b_prev, W_next in ((b1, W2), (b2, W3), (b3, W4)):
        parts = _sc_aggregate(g, src3, dst3, zerosF)
        g = _tc_mid(parts, g, dinv, b_prev.reshape(1, F), W_next)

    parts = _sc_aggregate(g, src3, dst3, zerosF)
    y = _tc_out(parts, g, dinv, b4.reshape(1, F), Wout, bout.reshape(1, C))
    return y[:N]


# speedup vs baseline: 3.3522x; 1.4285x over previous
import functools

import jax
import jax.numpy as jnp
from jax import lax
from jax.experimental import pallas as pl
from jax.experimental.pallas import tpu as pltpu
from jax.experimental.pallas import tpu_sc as plsc

N = 10000
F = 128
C = 16
E = 320000
NC, NS = 2, 16
NW = NC * NS

CHUNK = 128
NCHUNK = 80
EPW = NCHUNK * CHUNK
EPAD = NW * EPW

NPAD = 10240
STRIPE = NPAD // NS
ZROWS = 128

_vector_mesh = plsc.VectorSubcoreMesh(core_axis_name="c", subcore_axis_name="s")


@functools.partial(
    pl.kernel,
    mesh=_vector_mesh,
    out_type=jax.ShapeDtypeStruct((NC, NPAD, F), jnp.float32),
    scratch_types=[
        pltpu.VMEM((CHUNK,), jnp.int32),
        pltpu.VMEM((CHUNK,), jnp.int32),
        pltpu.VMEM((CHUNK,), jnp.int32),
        pltpu.VMEM((CHUNK,), jnp.int32),
        pltpu.VMEM((CHUNK, F), jnp.float32),
        pltpu.VMEM((CHUNK, F), jnp.float32),
        pltpu.VMEM_SHARED((NPAD, F), jnp.float32),
        pltpu.SemaphoreType.DMA,
        pltpu.SemaphoreType.DMA,
        pltpu.SemaphoreType.DMA,
        pltpu.SemaphoreType.DMA,
        pltpu.SemaphoreType.DMA,
        pltpu.SemaphoreType.DMA,
    ],
)
def _sc_aggregate(g_hbm, src_hbm, dst_hbm, zeros_hbm, out_hbm,
                  srcA, srcB, dstA, dstB, rowsA, rowsB, acc_sh,
                  semSA, semSB, semDA, semDB, semGA, semGB):
    cid = lax.axis_index("c")
    sid = lax.axis_index("s")
    wid = cid * NS + sid

    base = sid * STRIPE
    pltpu.sync_copy(zeros_hbm, acc_sh.at[pl.ds(base, STRIPE)])
    plsc.subcore_barrier()

    pltpu.async_copy(src_hbm.at[wid, 0], srcA, semSA)
    pltpu.async_copy(dst_hbm.at[wid, 0], dstA, semDA)
    pltpu.make_async_copy(src_hbm.at[wid, 0], srcA, semSA).wait()
    pltpu.async_copy(g_hbm.at[srcA], rowsA, semGA)

    @pl.loop(0, NCHUNK // 2)
    def _(i):
        j = i * 2
        pltpu.async_copy(src_hbm.at[wid, j + 1], srcB, semSB)
        pltpu.async_copy(dst_hbm.at[wid, j + 1], dstB, semDB)
        pltpu.make_async_copy(g_hbm.at[srcA], rowsA, semGA).wait()
        pltpu.make_async_copy(src_hbm.at[wid, j + 1], srcB, semSB).wait()
        pltpu.async_copy(g_hbm.at[srcB], rowsB, semGB)
        pltpu.make_async_copy(dst_hbm.at[wid, j], dstA, semDA).wait()
        pltpu.sync_copy(rowsA, acc_sh.at[dstA], add=True)

        @pl.when(j + 2 < NCHUNK)
        def _():
            pltpu.async_copy(src_hbm.at[wid, j + 2], srcA, semSA)
            pltpu.async_copy(dst_hbm.at[wid, j + 2], dstA, semDA)

        pltpu.make_async_copy(g_hbm.at[srcB], rowsB, semGB).wait()

        @pl.when(j + 2 < NCHUNK)
        def _():
            pltpu.make_async_copy(src_hbm.at[wid, j + 2], srcA, semSA).wait()
            pltpu.async_copy(g_hbm.at[srcA], rowsA, semGA)

        pltpu.make_async_copy(dst_hbm.at[wid, j + 1], dstB, semDB).wait()
        pltpu.sync_copy(rowsB, acc_sh.at[dstB], add=True)

    plsc.subcore_barrier()
    pltpu.sync_copy(acc_sh.at[pl.ds(base, STRIPE)],
                    out_hbm.at[cid, pl.ds(base, STRIPE)])


@functools.partial(
    pl.kernel,
    mesh=_vector_mesh,
    out_type=jax.ShapeDtypeStruct((NC, NPAD, F), jnp.float32),
    scratch_types=[
        pltpu.VMEM((CHUNK,), jnp.int32),
        pltpu.VMEM((CHUNK, F), jnp.float32),
        pltpu.VMEM_SHARED((NPAD, F), jnp.float32),
    ],
)
def _sc_degree(dst_hbm, ones_hbm, zeros_hbm, out_hbm,
               dst_v, ones_v, acc_sh):
    cid = lax.axis_index("c")
    sid = lax.axis_index("s")
    wid = cid * NS + sid
    base = sid * STRIPE
    pltpu.sync_copy(zeros_hbm, acc_sh.at[pl.ds(base, STRIPE)])
    pltpu.sync_copy(ones_hbm, ones_v)
    plsc.subcore_barrier()

    @pl.loop(0, NCHUNK)
    def _(j):
        pltpu.sync_copy(dst_hbm.at[wid, j], dst_v)
        pltpu.sync_copy(ones_v, acc_sh.at[dst_v], add=True)

    plsc.subcore_barrier()
    pltpu.sync_copy(acc_sh.at[pl.ds(base, STRIPE)],
                    out_hbm.at[cid, pl.ds(base, STRIPE)])


BLK = 512


def _tc_first_body(deg_ref, x_ref, w_ref, g_ref, dinv_ref):
    deg = deg_ref[0] + deg_ref[1] + 1.0
    dinv = lax.rsqrt(deg)
    h = jnp.dot(x_ref[...], w_ref[...], preferred_element_type=jnp.float32)
    g_ref[...] = h * dinv[:, :1]
    dinv_ref[...] = dinv[:, :C]


def _tc_mid_body(parts_ref, g_ref, dinv_ref, b_ref, w_ref, gout_ref):
    dinv = dinv_ref[:, :1]
    s = parts_ref[0] + parts_ref[1] + g_ref[...]
    xl = jnp.maximum(s * dinv + b_ref[...], 0.0)
    gout_ref[...] = jnp.dot(xl, w_ref[...],
                            preferred_element_type=jnp.float32) * dinv


def _tc_out_body(parts_ref, g_ref, dinv_ref, b_ref, w_ref, bout_ref, y_ref):
    dinv = dinv_ref[:, :1]
    s = parts_ref[0] + parts_ref[1] + g_ref[...]
    xl = jnp.maximum(s * dinv + b_ref[...], 0.0)
    logits = jnp.dot(xl, w_ref[...],
                     preferred_element_type=jnp.float32) + bout_ref[...]
    m = jnp.max(logits, axis=1, keepdims=True)
    e = jnp.exp(logits - m)
    y_ref[...] = e / jnp.sum(e, axis=1, keepdims=True)


_GRID = (NPAD // BLK,)
_spec_rowsF = pl.BlockSpec((BLK, F), lambda i: (i, 0))
_spec_rowsC = pl.BlockSpec((BLK, C), lambda i: (i, 0))
_spec_partsF = pl.BlockSpec((NC, BLK, F), lambda i: (0, i, 0))
_spec_partsC = pl.BlockSpec((NC, BLK, C), lambda i: (0, i, 0))
_spec_wFF = pl.BlockSpec((F, F), lambda i: (0, 0))
_spec_wFC = pl.BlockSpec((F, C), lambda i: (0, 0))
_spec_bF = pl.BlockSpec((1, F), lambda i: (0, 0))
_spec_bC = pl.BlockSpec((1, C), lambda i: (0, 0))

_tc_first = pl.pallas_call(
    _tc_first_body,
    grid=_GRID,
    in_specs=[_spec_partsF, _spec_rowsF, _spec_wFF],
    out_specs=[_spec_rowsF, _spec_rowsC],
    out_shape=[jax.ShapeDtypeStruct((NPAD, F), jnp.float32),
               jax.ShapeDtypeStruct((NPAD, C), jnp.float32)],
)

_tc_mid = pl.pallas_call(
    _tc_mid_body,
    grid=_GRID,
    in_specs=[_spec_partsF, _spec_rowsF, _spec_rowsC, _spec_bF, _spec_wFF],
    out_specs=_spec_rowsF,
    out_shape=jax.ShapeDtypeStruct((NPAD, F), jnp.float32),
)

_tc_out = pl.pallas_call(
    _tc_out_body,
    grid=_GRID,
    in_specs=[_spec_partsF, _spec_rowsF, _spec_rowsC, _spec_bF, _spec_wFC,
              _spec_bC],
    out_specs=_spec_rowsC,
    out_shape=jax.ShapeDtypeStruct((NPAD, C), jnp.float32),
)


def kernel(x, edge_index, W1, b1, W2, b2, W3, b3, W4, b4, Wout, bout):
    ei = edge_index.astype(jnp.int32)
    pad = EPAD - E
    spare = N + jnp.arange(pad, dtype=jnp.int32) % (NPAD - N)
    src = jnp.concatenate([ei[0], spare])
    dst = jnp.concatenate([ei[1], spare])
    src3 = src.reshape(NW, NCHUNK, CHUNK)
    dst3 = dst.reshape(NW, NCHUNK, CHUNK)

    x_pad = jnp.pad(x, ((0, NPAD - N), (0, 0)))
    zerosF = jnp.zeros((STRIPE, F), jnp.float32)
    onesF = jnp.ones((CHUNK, F), jnp.float32)

    deg_parts = _sc_degree(dst3, onesF, zerosF)
    g, dinv = _tc_first(deg_parts, x_pad, W1)

    for b_prev, W_next in ((b1, W2), (b2, W3), (b3, W4)):
        parts = _sc_aggregate(g, src3, dst3, zerosF)
        g = _tc_mid(parts, g, dinv, b_prev.reshape(1, F), W_next)

    parts = _sc_aggregate(g, src3, dst3, zerosF)
    y = _tc_out(parts, g, dinv, b4.reshape(1, F), Wout, bout.reshape(1, C))
    return y[:N]


# speedup vs baseline: 3.4967x; 1.0431x over previous
import functools

import jax
import jax.numpy as jnp
from jax import lax
from jax.experimental import pallas as pl
from jax.experimental.pallas import tpu as pltpu
from jax.experimental.pallas import tpu_sc as plsc

N = 10000
F = 128
C = 16
E = 320000
NC, NS = 2, 16
NW = NC * NS

CHUNK = 128
NCHUNK = 80
EPW = NCHUNK * CHUNK
EPAD = NW * EPW

NPAD = 10240
STRIPE = NPAD // NS
ZROWS = 128

_vector_mesh = plsc.VectorSubcoreMesh(core_axis_name="c", subcore_axis_name="s")


@functools.partial(
    pl.kernel,
    mesh=_vector_mesh,
    out_type=jax.ShapeDtypeStruct((NC, NPAD, F), jnp.float32),
    scratch_types=[
        pltpu.VMEM((CHUNK,), jnp.int32),
        pltpu.VMEM((CHUNK,), jnp.int32),
        pltpu.VMEM((CHUNK,), jnp.int32),
        pltpu.VMEM((CHUNK,), jnp.int32),
        pltpu.VMEM((CHUNK, F), jnp.float32),
        pltpu.VMEM((CHUNK, F), jnp.float32),
        pltpu.VMEM_SHARED((NPAD, F), jnp.float32),
        pltpu.SemaphoreType.DMA,
        pltpu.SemaphoreType.DMA,
        pltpu.SemaphoreType.DMA,
        pltpu.SemaphoreType.DMA,
        pltpu.SemaphoreType.DMA,
        pltpu.SemaphoreType.DMA,
    ],
)
def _sc_aggregate(g_hbm, src_hbm, dst_hbm, zeros_hbm, out_hbm,
                  srcA, srcB, dstA, dstB, rowsA, rowsB, acc_sh,
                  semSA, semSB, semDA, semDB, semGA, semGB):
    cid = lax.axis_index("c")
    sid = lax.axis_index("s")
    wid = cid * NS + sid

    base = sid * STRIPE
    pltpu.sync_copy(zeros_hbm, acc_sh.at[pl.ds(base, STRIPE)])
    plsc.subcore_barrier()

    pltpu.async_copy(src_hbm.at[wid, 0], srcA, semSA)
    pltpu.async_copy(dst_hbm.at[wid, 0], dstA, semDA)
    pltpu.make_async_copy(src_hbm.at[wid, 0], srcA, semSA).wait()
    pltpu.async_copy(g_hbm.at[srcA], rowsA, semGA)

    @pl.loop(0, NCHUNK // 2)
    def _(i):
        j = i * 2
        pltpu.async_copy(src_hbm.at[wid, j + 1], srcB, semSB)
        pltpu.async_copy(dst_hbm.at[wid, j + 1], dstB, semDB)
        pltpu.make_async_copy(g_hbm.at[srcA], rowsA, semGA).wait()
        pltpu.make_async_copy(src_hbm.at[wid, j + 1], srcB, semSB).wait()
        pltpu.async_copy(g_hbm.at[srcB], rowsB, semGB)
        pltpu.make_async_copy(dst_hbm.at[wid, j], dstA, semDA).wait()
        pltpu.sync_copy(rowsA, acc_sh.at[dstA], add=True)

        @pl.when(j + 2 < NCHUNK)
        def _():
            pltpu.async_copy(src_hbm.at[wid, j + 2], srcA, semSA)
            pltpu.async_copy(dst_hbm.at[wid, j + 2], dstA, semDA)

        pltpu.make_async_copy(g_hbm.at[srcB], rowsB, semGB).wait()

        @pl.when(j + 2 < NCHUNK)
        def _():
            pltpu.make_async_copy(src_hbm.at[wid, j + 2], srcA, semSA).wait()
            pltpu.async_copy(g_hbm.at[srcA], rowsA, semGA)

        pltpu.make_async_copy(dst_hbm.at[wid, j + 1], dstB, semDB).wait()
        pltpu.sync_copy(rowsB, acc_sh.at[dstB], add=True)

    plsc.subcore_barrier()
    pltpu.sync_copy(acc_sh.at[pl.ds(base, STRIPE)],
                    out_hbm.at[cid, pl.ds(base, STRIPE)])


@functools.partial(
    pl.kernel,
    mesh=_vector_mesh,
    out_type=jax.ShapeDtypeStruct((NC, NPAD, F), jnp.float32),
    scratch_types=[
        pltpu.VMEM((CHUNK,), jnp.int32),
        pltpu.VMEM((CHUNK,), jnp.int32),
        pltpu.VMEM((CHUNK, F), jnp.float32),
        pltpu.VMEM_SHARED((NPAD, F), jnp.float32),
        pltpu.SemaphoreType.DMA,
        pltpu.SemaphoreType.DMA,
    ],
)
def _sc_degree(dst_hbm, ones_hbm, zeros_hbm, out_hbm,
               dstA, dstB, ones_v, acc_sh, semDA, semDB):
    cid = lax.axis_index("c")
    sid = lax.axis_index("s")
    wid = cid * NS + sid
    base = sid * STRIPE
    pltpu.sync_copy(zeros_hbm, acc_sh.at[pl.ds(base, STRIPE)])
    pltpu.sync_copy(ones_hbm, ones_v)
    plsc.subcore_barrier()

    pltpu.async_copy(dst_hbm.at[wid, 0], dstA, semDA)

    @pl.loop(0, NCHUNK // 2)
    def _(i):
        j = i * 2
        pltpu.async_copy(dst_hbm.at[wid, j + 1], dstB, semDB)
        pltpu.make_async_copy(dst_hbm.at[wid, j], dstA, semDA).wait()
        pltpu.sync_copy(ones_v, acc_sh.at[dstA], add=True)

        @pl.when(j + 2 < NCHUNK)
        def _():
            pltpu.async_copy(dst_hbm.at[wid, j + 2], dstA, semDA)

        pltpu.make_async_copy(dst_hbm.at[wid, j + 1], dstB, semDB).wait()
        pltpu.sync_copy(ones_v, acc_sh.at[dstB], add=True)

    plsc.subcore_barrier()
    pltpu.sync_copy(acc_sh.at[pl.ds(base, STRIPE)],
                    out_hbm.at[cid, pl.ds(base, STRIPE)])


BLK = 512


def _tc_first_body(deg_ref, x_ref, w_ref, g_ref, dinv_ref):
    deg = deg_ref[0] + deg_ref[1] + 1.0
    dinv = lax.rsqrt(deg)
    h = jnp.dot(x_ref[...], w_ref[...], preferred_element_type=jnp.float32)
    g_ref[...] = h * dinv[:, :1]
    dinv_ref[...] = dinv[:, :C]


def _tc_mid_body(parts_ref, g_ref, dinv_ref, b_ref, w_ref, gout_ref):
    dinv = dinv_ref[:, :1]
    s = parts_ref[0] + parts_ref[1] + g_ref[...]
    xl = jnp.maximum(s * dinv + b_ref[...], 0.0)
    gout_ref[...] = jnp.dot(xl, w_ref[...],
                            preferred_element_type=jnp.float32) * dinv


def _tc_out_body(parts_ref, g_ref, dinv_ref, b_ref, w_ref, bout_ref, y_ref):
    dinv = dinv_ref[:, :1]
    s = parts_ref[0] + parts_ref[1] + g_ref[...]
    xl = jnp.maximum(s * dinv + b_ref[...], 0.0)
    logits = jnp.dot(xl, w_ref[...],
                     preferred_element_type=jnp.float32) + bout_ref[...]
    m = jnp.max(logits, axis=1, keepdims=True)
    e = jnp.exp(logits - m)
    y_ref[...] = e / jnp.sum(e, axis=1, keepdims=True)


_GRID = (NPAD // BLK,)
_spec_rowsF = pl.BlockSpec((BLK, F), lambda i: (i, 0))
_spec_rowsC = pl.BlockSpec((BLK, C), lambda i: (i, 0))
_spec_partsF = pl.BlockSpec((NC, BLK, F), lambda i: (0, i, 0))
_spec_partsC = pl.BlockSpec((NC, BLK, C), lambda i: (0, i, 0))
_spec_wFF = pl.BlockSpec((F, F), lambda i: (0, 0))
_spec_wFC = pl.BlockSpec((F, C), lambda i: (0, 0))
_spec_bF = pl.BlockSpec((1, F), lambda i: (0, 0))
_spec_bC = pl.BlockSpec((1, C), lambda i: (0, 0))

_tc_first = pl.pallas_call(
    _tc_first_body,
    grid=_GRID,
    in_specs=[_spec_partsF, _spec_rowsF, _spec_wFF],
    out_specs=[_spec_rowsF, _spec_rowsC],
    out_shape=[jax.ShapeDtypeStruct((NPAD, F), jnp.float32),
               jax.ShapeDtypeStruct((NPAD, C), jnp.float32)],
)

_tc_mid = pl.pallas_call(
    _tc_mid_body,
    grid=_GRID,
    in_specs=[_spec_partsF, _spec_rowsF, _spec_rowsC, _spec_bF, _spec_wFF],
    out_specs=_spec_rowsF,
    out_shape=jax.ShapeDtypeStruct((NPAD, F), jnp.float32),
)

_tc_out = pl.pallas_call(
    _tc_out_body,
    grid=_GRID,
    in_specs=[_spec_partsF, _spec_rowsF, _spec_rowsC, _spec_bF, _spec_wFC,
              _spec_bC],
    out_specs=_spec_rowsC,
    out_shape=jax.ShapeDtypeStruct((NPAD, C), jnp.float32),
)


def kernel(x, edge_index, W1, b1, W2, b2, W3, b3, W4, b4, Wout, bout):
    ei = edge_index.astype(jnp.int32)
    pad = EPAD - E
    spare = N + jnp.arange(pad, dtype=jnp.int32) % (NPAD - N)
    src = jnp.concatenate([ei[0], spare])
    dst = jnp.concatenate([ei[1], spare])
    src3 = src.reshape(NW, NCHUNK, CHUNK)
    dst3 = dst.reshape(NW, NCHUNK, CHUNK)

    x_pad = jnp.pad(x, ((0, NPAD - N), (0, 0)))
    zerosF = jnp.zeros((STRIPE, F), jnp.float32)
    onesF = jnp.ones((CHUNK, F), jnp.float32)

    deg_parts = _sc_degree(dst3, onesF, zerosF)
    g, dinv = _tc_first(deg_parts, x_pad, W1)

    for b_prev, W_next in ((b1, W2), (b2, W3), (b3, W4)):
        parts = _sc_aggregate(g, src3, dst3, zerosF)
        g = _tc_mid(parts, g, dinv, b_prev.reshape(1, F), W_next)

    parts = _sc_aggregate(g, src3, dst3, zerosF)
    y = _tc_out(parts, g, dinv, b4.reshape(1, F), Wout, bout.reshape(1, C))
    return y[:N]
